# Initial kernel scaffold; baseline (speedup 1.0000x reference)
import jax
import jax.numpy as jnp
from jax.experimental import pallas as pl


def kernel(x_with_meta, W1, b1, W2, b2, W3, b3, Wb1, bb1, Wb2, bb2):
    raise NotImplementedError("write your pallas kernel here")


# speedup vs baseline: 14.7273x; 14.7273x over previous
import functools

import jax
import jax.numpy as jnp
from jax import lax
from jax.experimental import pallas as pl

G = 512
CH = 512


def _body(x_ref, gids_ref, mask_ref, w1_ref, b1_ref, w2_ref, b2_ref,
          w3_ref, b3_ref, wb1_ref, bb1_ref, wb2_ref, bb2_ref, out_ref):
    C = x_ref.shape[1]

    feats = x_ref[0]
    h1 = jnp.maximum(
        jnp.dot(feats, w1_ref[...], preferred_element_type=jnp.float32)
        + b1_ref[...], 0.0)
    h2 = jnp.maximum(
        jnp.dot(h1, w2_ref[...], preferred_element_type=jnp.float32)
        + b2_ref[...], 0.0)
    raw = lax.dot_general(w3_ref[...], h2,
                          (((1,), (1,)), ((), ())),
                          preferred_element_type=jnp.float32)
    raw = raw + b3_ref[0:1, 0:1]

    gids = gids_ref[0]
    msk = mask_ref[0]
    grp = (msk > 0.0) & (gids >= 0)

    gsub = lax.broadcasted_iota(jnp.int32, (G, CH), 0)

    smax = jnp.full((G, 1), -1e9, jnp.float32)
    ssum = jnp.zeros((G, 1), jnp.float32)
    cnt = jnp.zeros((G, 1), jnp.float32)
    for k in range(C // CH):
        sl = slice(k * CH, (k + 1) * CH)
        cmp = (gsub == gids[:, sl]) & grp[:, sl]
        raw_k = raw[:, sl]
        smax = jnp.maximum(
            smax, jnp.max(jnp.where(cmp, raw_k, -1e9), axis=1, keepdims=True))
        ssum = ssum + jnp.sum(jnp.where(cmp, raw_k, 0.0), axis=1, keepdims=True)
        cnt = cnt + jnp.sum(cmp.astype(jnp.float32), axis=1, keepdims=True)

    has = cnt > 0.0
    smax0 = jnp.where(has, smax, 0.0)
    mean = ssum / jnp.maximum(cnt, 1.0)

    hb = jnp.maximum(
        smax0 * wb1_ref[0:1, :] + mean * wb1_ref[1:2, :] + cnt * wb1_ref[2:3, :]
        + bb1_ref[...], 0.0)
    bias = jnp.sum(hb * wb2_ref[...], axis=1, keepdims=True) + bb2_ref[0:1, 0:1]

    t = 0.5 * smax0 - mean + bias
    cgt1 = cnt > 1.0

    for k in range(C // CH):
        sl = slice(k * CH, (k + 1) * CH)
        cmp = (gsub == gids[:, sl]) & grp[:, sl]
        raw_k = raw[:, sl]
        msk_k = msk[:, sl]
        t_e = jnp.sum(jnp.where(cmp, t, 0.0), axis=0, keepdims=True)
        apply_k = jnp.any(cmp & cgt1, axis=0, keepdims=True)
        base = jnp.where(msk_k <= 0.0, -1e9, raw_k)
        out_ref[0, 0:1, sl] = jnp.where(apply_k, 1.5 * raw_k + t_e, base)


def kernel(x_with_meta, W1, b1, W2, b2, W3, b3, Wb1, bb1, Wb2, bb2):
    B, C, Fp2 = x_with_meta.shape
    F = Fp2 - 2
    H1 = W1.shape[1]
    H2 = W2.shape[1]

    gids3 = x_with_meta[:, :, F].astype(jnp.int32).reshape(B, 1, C)
    mask = x_with_meta[:, :, F + 1]
    mask3 = mask.reshape(B, 1, C)

    fair3 = pl.pallas_call(
        _body,
        grid=(B,),
        in_specs=[
            pl.BlockSpec((1, C, F), lambda b: (b, 0, 0)),
            pl.BlockSpec((1, 1, C), lambda b: (b, 0, 0)),
            pl.BlockSpec((1, 1, C), lambda b: (b, 0, 0)),
            pl.BlockSpec((F, H1), lambda b: (0, 0)),
            pl.BlockSpec((1, H1), lambda b: (0, 0)),
            pl.BlockSpec((H1, H2), lambda b: (0, 0)),
            pl.BlockSpec((1, H2), lambda b: (0, 0)),
            pl.BlockSpec((1, H2), lambda b: (0, 0)),
            pl.BlockSpec((1, 1), lambda b: (0, 0)),
            pl.BlockSpec((3, 32), lambda b: (0, 0)),
            pl.BlockSpec((1, 32), lambda b: (0, 0)),
            pl.BlockSpec((1, 32), lambda b: (0, 0)),
            pl.BlockSpec((1, 1), lambda b: (0, 0)),
        ],
        out_specs=pl.BlockSpec((1, 1, C), lambda b: (b, 0, 0)),
        out_shape=jax.ShapeDtypeStruct((B, 1, C), jnp.float32),
    )(
        x_with_meta, gids3, mask3,
        W1, b1.reshape(1, H1), W2, b2.reshape(1, H2),
        W3.reshape(1, H2), b3.reshape(1, 1),
        Wb1, bb1.reshape(1, 32), Wb2.reshape(1, 32), bb2.reshape(1, 1),
    )
    return fair3.reshape(B, C), mask


# speedup vs baseline: 19.0015x; 1.2902x over previous
import jax
import jax.numpy as jnp
from jax import lax
from jax.experimental import pallas as pl
from jax.experimental.pallas import tpu as pltpu
from jax.experimental.pallas import tpu_sc as plsc

B, C, F, G = 16, 4096, 128, 512
H1, H2 = 256, 128
NLANE = 16
WBPAD = 176
NEG = -1e9


def _mlp_body(x_ref, w1_ref, b1_ref, w2_ref, b2_ref, w3_ref, b3_ref, out_ref):
    feats = x_ref[0]
    h1 = jnp.maximum(
        jnp.dot(feats, w1_ref[...], preferred_element_type=jnp.float32)
        + b1_ref[...], 0.0)
    h2 = jnp.maximum(
        jnp.dot(h1, w2_ref[...], preferred_element_type=jnp.float32)
        + b2_ref[...], 0.0)
    raw = lax.dot_general(w3_ref[...], h2,
                          (((1,), (1,)), ((), ())),
                          preferred_element_type=jnp.float32)
    out_ref[0] = raw + b3_ref[0:1, 0:1]


def _sc_body(raw_h, gid_h, msk_h, wb_h, binit_h, out_h,
             raw_v, gid_v, msk_v, out_v, wb_v, cnt_v, sum_v, max_v, t_v,
             wb_s, sem):
    c = lax.axis_index("c")
    s = lax.axis_index("s")

    @pl.when(s < 8)
    def _work():
        batch = c * 8 + s
        base = pl.multiple_of(batch * C, C)

        cps = [
            pltpu.async_copy(raw_h.at[pl.ds(base, C)], raw_v, sem),
            pltpu.async_copy(gid_h.at[pl.ds(base, C)], gid_v, sem),
            pltpu.async_copy(msk_h.at[pl.ds(base, C)], msk_v, sem),
            pltpu.async_copy(wb_h, wb_v, sem),
            pltpu.async_copy(binit_h.at[pl.ds(0, NLANE * G)], cnt_v, sem),
            pltpu.async_copy(binit_h.at[pl.ds(NLANE * G, NLANE * G)], sum_v, sem),
            pltpu.async_copy(binit_h.at[pl.ds(2 * NLANE * G, NLANE * G)], max_v, sem),
        ]
        for cp in cps:
            cp.wait()

        lanei = lax.iota(jnp.int32, NLANE)
        ones = jnp.ones((NLANE,), jnp.float32)

        for blk in range(WBPAD // NLANE):
            v = wb_v[pl.ds(blk * NLANE, NLANE)]
            for l in range(NLANE):
                i = blk * NLANE + l
                if i > 160:
                    break
                wb_s[i] = jnp.max(jnp.where(lanei == l, v, jnp.float32(-3.4e38)))

        def _accum(i, carry):
            off = pl.multiple_of(i * NLANE, NLANE)
            g = gid_v[pl.ds(off, NLANE)]
            v = raw_v[pl.ds(off, NLANE)]
            m = msk_v[pl.ds(off, NLANE)]
            grp = (m > 0.0) & (g >= 0)
            idx = lanei * G + jnp.where(grp, g, 0)
            cur = plsc.load_gather(max_v, [idx], mask=grp)
            plsc.store_scatter(max_v, [idx], jnp.maximum(cur, v), mask=grp)
            plsc.addupdate_scatter(cnt_v, [idx], ones, mask=grp)
            plsc.addupdate_scatter(sum_v, [idx], v, mask=grp)
            return carry
        lax.fori_loop(0, C // NLANE, _accum, 0)

        def _reduce(blk, carry):
            off = pl.multiple_of(blk * NLANE, NLANE)
            cc = cnt_v[pl.ds(off, NLANE)]
            ss = sum_v[pl.ds(off, NLANE)]
            mm = max_v[pl.ds(off, NLANE)]
            for r in range(1, NLANE):
                o2 = pl.multiple_of(r * G + blk * NLANE, NLANE)
                cc = cc + cnt_v[pl.ds(o2, NLANE)]
                ss = ss + sum_v[pl.ds(o2, NLANE)]
                mm = jnp.maximum(mm, max_v[pl.ds(o2, NLANE)])
            cnt_v[pl.ds(off, NLANE)] = cc
            sum_v[pl.ds(off, NLANE)] = ss
            max_v[pl.ds(off, NLANE)] = mm
            return carry
        lax.fori_loop(0, G // NLANE, _reduce, 0)

        def _bias(blk, carry):
            off = pl.multiple_of(blk * NLANE, NLANE)
            cc = cnt_v[pl.ds(off, NLANE)]
            ss = sum_v[pl.ds(off, NLANE)]
            mm = max_v[pl.ds(off, NLANE)]
            mx0 = jnp.where(cc > 0.0, mm, 0.0)
            mean = ss / jnp.maximum(cc, 1.0)
            acc = jnp.zeros((NLANE,), jnp.float32)
            for j in range(32):
                h = mx0 * wb_s[j] + mean * wb_s[32 + j] + cc * wb_s[64 + j] \
                    + wb_s[96 + j]
                acc = acc + jnp.maximum(h, 0.0) * wb_s[128 + j]
            t_v[pl.ds(off, NLANE)] = 0.5 * mx0 - mean + (acc + wb_s[160])
            return carry
        lax.fori_loop(0, G // NLANE, _bias, 0)

        def _combine(i, carry):
            off = pl.multiple_of(i * NLANE, NLANE)
            g = gid_v[pl.ds(off, NLANE)]
            v = raw_v[pl.ds(off, NLANE)]
            m = msk_v[pl.ds(off, NLANE)]
            grp = (m > 0.0) & (g >= 0)
            gg = jnp.where(grp, g, 0)
            te = plsc.load_gather(t_v, [gg], mask=grp)
            ce = plsc.load_gather(cnt_v, [gg], mask=grp)
            ap = grp & (ce > 1.0)
            fair = jnp.where(ap, 1.5 * v + te,
                             jnp.where(m <= 0.0, jnp.float32(NEG), v))
            out_v[pl.ds(off, NLANE)] = fair
            return carry
        lax.fori_loop(0, C // NLANE, _combine, 0)

        pltpu.sync_copy(out_v, out_h.at[pl.ds(base, C)])


_sc_post = pl.kernel(
    _sc_body,
    out_type=jax.ShapeDtypeStruct((B * C,), jnp.float32),
    mesh=plsc.VectorSubcoreMesh(core_axis_name="c", subcore_axis_name="s"),
    compiler_params=pltpu.CompilerParams(needs_layout_passes=False),
    scratch_types=[
        pltpu.VMEM((C,), jnp.float32),
        pltpu.VMEM((C,), jnp.int32),
        pltpu.VMEM((C,), jnp.float32),
        pltpu.VMEM((C,), jnp.float32),
        pltpu.VMEM((WBPAD,), jnp.float32),
        pltpu.VMEM((NLANE * G,), jnp.float32),
        pltpu.VMEM((NLANE * G,), jnp.float32),
        pltpu.VMEM((NLANE * G,), jnp.float32),
        pltpu.VMEM((G,), jnp.float32),
        pltpu.SMEM((WBPAD,), jnp.float32),
        pltpu.SemaphoreType.DMA,
    ],
)


def kernel(x_with_meta, W1, b1, W2, b2, W3, b3, Wb1, bb1, Wb2, bb2):
    mask = x_with_meta[:, :, F + 1]
    gid_flat = x_with_meta[:, :, F].astype(jnp.int32).reshape(B * C)
    msk_flat = mask.reshape(B * C)

    raw3 = pl.pallas_call(
        _mlp_body,
        grid=(B,),
        in_specs=[
            pl.BlockSpec((1, C, F), lambda b: (b, 0, 0)),
            pl.BlockSpec((F, H1), lambda b: (0, 0)),
            pl.BlockSpec((1, H1), lambda b: (0, 0)),
            pl.BlockSpec((H1, H2), lambda b: (0, 0)),
            pl.BlockSpec((1, H2), lambda b: (0, 0)),
            pl.BlockSpec((1, H2), lambda b: (0, 0)),
            pl.BlockSpec((1, 1), lambda b: (0, 0)),
        ],
        out_specs=pl.BlockSpec((1, 1, C), lambda b: (b, 0, 0)),
        out_shape=jax.ShapeDtypeStruct((B, 1, C), jnp.float32),
    )(
        x_with_meta, W1, b1.reshape(1, H1), W2, b2.reshape(1, H2),
        W3.reshape(1, H2), b3.reshape(1, 1),
    )

    wb = jnp.concatenate([
        Wb1[0], Wb1[1], Wb1[2], bb1, Wb2[:, 0], bb2,
        jnp.zeros((WBPAD - 161,), jnp.float32),
    ])
    binit = jnp.concatenate([
        jnp.zeros((2 * NLANE * G,), jnp.float32),
        jnp.full((NLANE * G,), NEG, jnp.float32),
    ])

    fair_flat = _sc_post(raw3.reshape(B * C), gid_flat, msk_flat, wb, binit)
    return fair_flat.reshape(B, C), mask
